# baseline (speedup 1.0000x reference)
import jax
import jax.numpy as jnp
from jax.experimental import pallas as pl


def _quantize_kernel(wq_ref, cb_ref, idx_ref, we_ref):
    wq = wq_ref[0]
    cb = cb_ref[0]
    B, D = wq.shape
    K = cb.shape[0]
    wq_sq = jnp.sum(wq * wq, axis=1)
    cb_sq = jnp.sum(cb * cb, axis=1)
    wq_p = jnp.concatenate([wq, jnp.zeros((B, 128 - D), jnp.float32)], axis=1)
    cb_p = jnp.concatenate([cb, jnp.zeros((K, 128 - D), jnp.float32)], axis=1)
    cross = jax.lax.dot_general(
        wq_p, cb_p, (((1,), (1,)), ((), ())),
        preferred_element_type=jnp.float32,
        precision=jax.lax.Precision.DEFAULT)
    d = (wq_sq[:, None] + cb_sq[None, :]) - 2.0 * cross
    idx = jnp.argmin(d, axis=1).astype(jnp.int32)
    idx_ref[0, 0] = idx
    kio = jax.lax.broadcasted_iota(jnp.int32, (B, K), 1)
    oh = (kio == idx[:, None]).astype(jnp.float32)
    we = jax.lax.dot_general(
        oh, cb, (((1,), (0,)), ((), ())),
        preferred_element_type=jnp.float32,
        precision=jax.lax.Precision.HIGHEST)
    we_ref[0] = we


def _onehot_kernel(idx_ref, oh_ref):
    row = idx_ref[0, 0]
    C = row.shape[0]
    K = oh_ref.shape[2]
    kio = jax.lax.broadcasted_iota(jnp.int32, (C, K), 1)
    oh_ref[0] = (kio == row[:, None]).astype(oh_ref.dtype)


def kernel(w_q, codebook):
    B, C, D = w_q.shape
    K = codebook.shape[1]
    wq_t = jnp.transpose(w_q, (1, 0, 2))
    idx_t, we_t = pl.pallas_call(
        _quantize_kernel,
        grid=(C,),
        in_specs=[
            pl.BlockSpec((1, B, D), lambda c: (c, 0, 0)),
            pl.BlockSpec((1, K, D), lambda c: (c, 0, 0)),
        ],
        out_specs=[
            pl.BlockSpec((1, 1, B), lambda c: (c, 0, 0)),
            pl.BlockSpec((1, B, D), lambda c: (c, 0, 0)),
        ],
        out_shape=[
            jax.ShapeDtypeStruct((C, 1, B), jnp.int32),
            jax.ShapeDtypeStruct((C, B, D), jnp.float32),
        ],
    )(wq_t, codebook)
    idx = jnp.transpose(idx_t[:, 0, :], (1, 0))
    w_e = jnp.transpose(we_t, (1, 0, 2))
    one_hot = pl.pallas_call(
        _onehot_kernel,
        grid=(B,),
        in_specs=[pl.BlockSpec((1, 1, C), lambda b: (b, 0, 0))],
        out_specs=pl.BlockSpec((1, C, K), lambda b: (b, 0, 0)),
        out_shape=jax.ShapeDtypeStruct((B, C, K), w_q.dtype),
    )(idx.reshape(B, 1, C))
    w = w_q + jax.lax.stop_gradient(w_e - w_q)
    return (w, w_e, idx, one_hot)


# speedup vs baseline: 1.3435x; 1.3435x over previous
import jax
import jax.numpy as jnp
from jax.experimental import pallas as pl


def _quantize_kernel(wq_ref, cb_ref, idx_ref, we_ref):
    wq = wq_ref[0]
    cb = cb_ref[0]
    B, D = wq.shape
    K = cb.shape[0]
    wq_sq = jnp.sum(wq * wq, axis=1)
    cb_sq = jnp.sum(cb * cb, axis=1)
    cross = jax.lax.dot_general(
        wq, cb, (((1,), (1,)), ((), ())),
        preferred_element_type=jnp.float32,
        precision=jax.lax.Precision.DEFAULT)
    d = (wq_sq[:, None] + cb_sq[None, :]) - 2.0 * cross
    idx = jnp.argmin(d, axis=1).astype(jnp.int32)
    idx_ref[0, 0] = idx
    kio = jax.lax.broadcasted_iota(jnp.int32, (B, K), 1)
    oh = (kio == idx[:, None]).astype(jnp.float32)
    we = jax.lax.dot_general(
        oh, cb, (((1,), (0,)), ((), ())),
        preferred_element_type=jnp.float32,
        precision=jax.lax.Precision.DEFAULT)
    we_ref[0] = we


def _onehot_kernel(idx_ref, oh_ref):
    row = idx_ref[0, 0]
    C = row.shape[0]
    K = oh_ref.shape[2]
    kio = jax.lax.broadcasted_iota(jnp.int32, (C, K), 1)
    oh_ref[0] = (kio == row[:, None]).astype(oh_ref.dtype)


def kernel(w_q, codebook):
    B, C, D = w_q.shape
    K = codebook.shape[1]
    wq_t = jnp.transpose(w_q, (1, 0, 2))
    idx_t, we_t = pl.pallas_call(
        _quantize_kernel,
        grid=(C,),
        in_specs=[
            pl.BlockSpec((1, B, D), lambda c: (c, 0, 0)),
            pl.BlockSpec((1, K, D), lambda c: (c, 0, 0)),
        ],
        out_specs=[
            pl.BlockSpec((1, 1, B), lambda c: (c, 0, 0)),
            pl.BlockSpec((1, B, D), lambda c: (c, 0, 0)),
        ],
        out_shape=[
            jax.ShapeDtypeStruct((C, 1, B), jnp.int32),
            jax.ShapeDtypeStruct((C, B, D), jnp.float32),
        ],
    )(wq_t, codebook)
    idx = jnp.transpose(idx_t[:, 0, :], (1, 0))
    w_e = jnp.transpose(we_t, (1, 0, 2))
    one_hot = pl.pallas_call(
        _onehot_kernel,
        grid=(B,),
        in_specs=[pl.BlockSpec((1, 1, C), lambda b: (b, 0, 0))],
        out_specs=pl.BlockSpec((1, C, K), lambda b: (b, 0, 0)),
        out_shape=jax.ShapeDtypeStruct((B, C, K), w_q.dtype),
    )(idx.reshape(B, 1, C))
    w = w_q + jax.lax.stop_gradient(w_e - w_q)
    return (w, w_e, idx, one_hot)


# speedup vs baseline: 1.3593x; 1.0118x over previous
import jax
import jax.numpy as jnp
from jax.experimental import pallas as pl


def _quantize_kernel(wq_ref, cb_ref, idx_ref, we_ref):
    NC = wq_ref.shape[0]
    for i in range(NC):
        wq = wq_ref[i]
        cb = cb_ref[i]
        B, D = wq.shape
        K = cb.shape[0]
        wq_sq = jnp.sum(wq * wq, axis=1)
        cb_sq = jnp.sum(cb * cb, axis=1)
        cross = jax.lax.dot_general(
            wq, cb, (((1,), (1,)), ((), ())),
            preferred_element_type=jnp.float32,
            precision=jax.lax.Precision.DEFAULT)
        d = (wq_sq[:, None] + cb_sq[None, :]) - 2.0 * cross
        idx = jnp.argmin(d, axis=1).astype(jnp.int32)
        idx_ref[i, 0] = idx
        kio = jax.lax.broadcasted_iota(jnp.int32, (B, K), 1)
        oh = (kio == idx[:, None]).astype(jnp.float32)
        we = jax.lax.dot_general(
            oh, cb, (((1,), (0,)), ((), ())),
            preferred_element_type=jnp.float32,
            precision=jax.lax.Precision.DEFAULT)
        we_ref[i] = we


def _onehot_kernel(idx_ref, oh_ref):
    row = idx_ref[0, 0]
    C = row.shape[0]
    K = oh_ref.shape[2]
    kio = jax.lax.broadcasted_iota(jnp.int32, (C, K), 1)
    oh_ref[0] = (kio == row[:, None]).astype(oh_ref.dtype)


def kernel(w_q, codebook):
    B, C, D = w_q.shape
    K = codebook.shape[1]
    wq_t = jnp.transpose(w_q, (1, 0, 2))
    NC = 4
    idx_t, we_t = pl.pallas_call(
        _quantize_kernel,
        grid=(C // NC,),
        in_specs=[
            pl.BlockSpec((NC, B, D), lambda c: (c, 0, 0)),
            pl.BlockSpec((NC, K, D), lambda c: (c, 0, 0)),
        ],
        out_specs=[
            pl.BlockSpec((NC, 1, B), lambda c: (c, 0, 0)),
            pl.BlockSpec((NC, B, D), lambda c: (c, 0, 0)),
        ],
        out_shape=[
            jax.ShapeDtypeStruct((C, 1, B), jnp.int32),
            jax.ShapeDtypeStruct((C, B, D), jnp.float32),
        ],
    )(wq_t, codebook)
    idx = jnp.transpose(idx_t[:, 0, :], (1, 0))
    w_e = jnp.transpose(we_t, (1, 0, 2))
    one_hot = pl.pallas_call(
        _onehot_kernel,
        grid=(B,),
        in_specs=[pl.BlockSpec((1, 1, C), lambda b: (b, 0, 0))],
        out_specs=pl.BlockSpec((1, C, K), lambda b: (b, 0, 0)),
        out_shape=jax.ShapeDtypeStruct((B, C, K), w_q.dtype),
    )(idx.reshape(B, 1, C))
    w = w_q + jax.lax.stop_gradient(w_e - w_q)
    return (w, w_e, idx, one_hot)


# speedup vs baseline: 3.2618x; 2.3996x over previous
import jax
import jax.numpy as jnp
from jax.experimental import pallas as pl


def _quantize_kernel(wq_ref, cbt_ref, idx_ref, wet_ref):
    NC = wq_ref.shape[0]
    for i in range(NC):
        wq = wq_ref[i]
        cbt = cbt_ref[i]
        B, D = wq.shape
        K = cbt.shape[1]
        wq_sq = jnp.sum(wq * wq, axis=1)
        cb_sq = jnp.sum(cbt * cbt, axis=0)
        cross = jax.lax.dot_general(
            wq, cbt, (((1,), (0,)), ((), ())),
            preferred_element_type=jnp.float32,
            precision=jax.lax.Precision.DEFAULT)
        d = (wq_sq[:, None] + cb_sq[None, :]) - 2.0 * cross
        idx = jnp.argmin(d, axis=1).astype(jnp.int32)
        idx_ref[i, 0] = idx
        kio_t = jax.lax.broadcasted_iota(jnp.int32, (K, B), 0)
        oh_t = (kio_t == idx[None, :]).astype(jnp.float32)
        wet = jax.lax.dot_general(
            cbt, oh_t, (((1,), (0,)), ((), ())),
            preferred_element_type=jnp.float32,
            precision=jax.lax.Precision.DEFAULT)
        wet_ref[i] = wet


def _onehot_kernel(idx_ref, oh_ref):
    row = idx_ref[0, 0]
    C = row.shape[0]
    K = oh_ref.shape[2]
    kio = jax.lax.broadcasted_iota(jnp.int32, (C, K), 1)
    oh_ref[0] = (kio == row[:, None]).astype(oh_ref.dtype)


def kernel(w_q, codebook):
    B, C, D = w_q.shape
    K = codebook.shape[1]
    wq_t = jnp.transpose(w_q, (1, 0, 2))
    cb_t = jnp.transpose(codebook, (0, 2, 1))
    NC = 4
    idx_t, we_t = pl.pallas_call(
        _quantize_kernel,
        grid=(C // NC,),
        in_specs=[
            pl.BlockSpec((NC, B, D), lambda c: (c, 0, 0)),
            pl.BlockSpec((NC, D, K), lambda c: (c, 0, 0)),
        ],
        out_specs=[
            pl.BlockSpec((NC, 1, B), lambda c: (c, 0, 0)),
            pl.BlockSpec((NC, D, B), lambda c: (c, 0, 0)),
        ],
        out_shape=[
            jax.ShapeDtypeStruct((C, 1, B), jnp.int32),
            jax.ShapeDtypeStruct((C, D, B), jnp.float32),
        ],
    )(wq_t, cb_t)
    idx = jnp.transpose(idx_t[:, 0, :], (1, 0))
    w_e = jnp.transpose(we_t, (2, 0, 1))
    one_hot = pl.pallas_call(
        _onehot_kernel,
        grid=(B,),
        in_specs=[pl.BlockSpec((1, 1, C), lambda b: (b, 0, 0))],
        out_specs=pl.BlockSpec((1, C, K), lambda b: (b, 0, 0)),
        out_shape=jax.ShapeDtypeStruct((B, C, K), w_q.dtype),
    )(idx.reshape(B, 1, C))
    w = w_q + jax.lax.stop_gradient(w_e - w_q)
    return (w, w_e, idx, one_hot)


# speedup vs baseline: 3.5037x; 1.0741x over previous
import jax
import jax.numpy as jnp
from jax.experimental import pallas as pl


def _quantize_kernel(wq_ref, cbt_ref, idx_ref, wet_ref):
    NC = wq_ref.shape[0]
    for i in range(NC):
        wq = wq_ref[i]
        cbt = cbt_ref[i]
        B, D = wq.shape
        K = cbt.shape[1]
        wq_sq = jnp.sum(wq * wq, axis=1)
        cb_sq = jnp.sum(cbt * cbt, axis=0)
        cross = jax.lax.dot_general(
            wq, cbt, (((1,), (0,)), ((), ())),
            preferred_element_type=jnp.float32,
            precision=jax.lax.Precision.DEFAULT)
        d = (wq_sq[:, None] + cb_sq[None, :]) - 2.0 * cross
        idx = jnp.argmin(d, axis=1).astype(jnp.int32)
        idx_ref[i, 0] = idx
        kio_t = jax.lax.broadcasted_iota(jnp.int32, (K, 1), 0)
        oh_t = (kio_t == idx[None, :]).astype(jnp.float32)
        wet = jax.lax.dot_general(
            cbt, oh_t, (((1,), (0,)), ((), ())),
            preferred_element_type=jnp.float32,
            precision=jax.lax.Precision.DEFAULT)
        wet_ref[i] = wet


def _onehot_kernel(idx_ref, oh_ref):
    NB = idx_ref.shape[0]
    C = idx_ref.shape[2]
    K = oh_ref.shape[2]
    kio = jax.lax.broadcasted_iota(jnp.int32, (C, K), 1)
    for j in range(NB):
        row = idx_ref[j, 0]
        oh_ref[j] = (kio == row[:, None]).astype(oh_ref.dtype)


def kernel(w_q, codebook):
    B, C, D = w_q.shape
    K = codebook.shape[1]
    wq_t = jnp.transpose(w_q, (1, 0, 2))
    cb_t = jnp.transpose(codebook, (0, 2, 1))
    NC = 8
    idx_t, we_t = pl.pallas_call(
        _quantize_kernel,
        grid=(C // NC,),
        in_specs=[
            pl.BlockSpec((NC, B, D), lambda c: (c, 0, 0)),
            pl.BlockSpec((NC, D, K), lambda c: (c, 0, 0)),
        ],
        out_specs=[
            pl.BlockSpec((NC, 1, B), lambda c: (c, 0, 0)),
            pl.BlockSpec((NC, D, B), lambda c: (c, 0, 0)),
        ],
        out_shape=[
            jax.ShapeDtypeStruct((C, 1, B), jnp.int32),
            jax.ShapeDtypeStruct((C, D, B), jnp.float32),
        ],
    )(wq_t, cb_t)
    idx = jnp.transpose(idx_t[:, 0, :], (1, 0))
    w_e = jnp.transpose(we_t, (2, 0, 1))
    NB = 8
    one_hot = pl.pallas_call(
        _onehot_kernel,
        grid=(B // NB,),
        in_specs=[pl.BlockSpec((NB, 1, C), lambda b: (b, 0, 0))],
        out_specs=pl.BlockSpec((NB, C, K), lambda b: (b, 0, 0)),
        out_shape=jax.ShapeDtypeStruct((B, C, K), w_q.dtype),
    )(idx.reshape(B, 1, C))
    w = w_q + jax.lax.stop_gradient(w_e - w_q)
    return (w, w_e, idx, one_hot)
